# baseline (speedup 1.0000x reference)
import functools

import jax
import jax.numpy as jnp
from jax import lax
from jax.experimental import pallas as pl
from jax.experimental.pallas import tpu as pltpu
from jax.experimental.pallas import tpu_sc as plsc

NC = 2
NS = 16
NW = NC * NS

LANES = 128
CHUNK = 104


@functools.lru_cache(maxsize=None)
def _make_gather(n_b: int, seq: int, dim: int):
    assert n_b % NW == 0
    b_per_w = n_b // NW
    assert b_per_w % 2 == 0
    assert seq == 2 * CHUNK - 8
    ovl = seq - CHUNK

    n_vec = dim // 16
    tail = dim - 16 * n_vec
    t_off = dim - 16

    mesh = plsc.VectorSubcoreMesh(core_axis_name="c", subcore_axis_name="s")

    @functools.partial(
        pl.kernel,
        mesh=mesh,
        out_type=jax.ShapeDtypeStruct((n_b, seq, dim), jnp.float32),
        scratch_types=[
            pltpu.VMEM((448,), jnp.int32),
            pltpu.VMEM((2, 2, CHUNK, LANES), jnp.float32),
            pltpu.VMEM((seq, dim), jnp.float32),
            pltpu.SemaphoreType.DMA,
        ],
        compiler_params=pltpu.CompilerParams(use_tc_tiling_on_sc=True),
    )
    def k(emb_hbm, xf_hbm, out_hbm, idx_v, rows_v, comp_v, sem_g):
        wid = lax.axis_index("s") * NC + lax.axis_index("c")
        b_base = wid * b_per_w

        def idx_slices(buf):
            base = buf * 224
            return (idx_v.at[pl.ds(base, CHUNK)],
                    idx_v.at[pl.ds(base + 112, CHUNK)])

        def fetch_idx(buf, b):
            ia, ib = idx_slices(buf)
            pltpu.sync_copy(xf_hbm.at[pl.ds(b * seq, CHUNK)], ia)
            pltpu.sync_copy(xf_hbm.at[pl.ds(b * seq + ovl, CHUNK)], ib)

        def fire(buf):
            ia, ib = idx_slices(buf)
            pltpu.async_copy(emb_hbm.at[ia], rows_v.at[buf, 0], sem_g)
            pltpu.async_copy(emb_hbm.at[ib], rows_v.at[buf, 1], sem_g)

        def drain(buf):
            ia, ib = idx_slices(buf)
            pltpu.make_async_copy(emb_hbm.at[ia], rows_v.at[buf, 0],
                                  sem_g).wait()
            pltpu.make_async_copy(emb_hbm.at[ib], rows_v.at[buf, 1],
                                  sem_g).wait()

        def compact(buf, half, lo, hi, shift):
            def row_body(t, carry):
                r = t - shift
                for v in range(n_vec):
                    comp_v[t, pl.ds(16 * v, 16)] = rows_v[
                        buf, half, r, pl.ds(16 * v, 16)
                    ]
                if tail:
                    comp_v[t, pl.ds(t_off, 16)] = rows_v[
                        buf, half, r, pl.ds(t_off, 16)
                    ]
                return carry

            lax.fori_loop(lo, hi, row_body, 0)

        def process(buf, b):
            compact(buf, 0, 0, CHUNK, 0)
            compact(buf, 1, CHUNK, seq, ovl)
            pltpu.sync_copy(comp_v, out_hbm.at[b])

        fetch_idx(0, b_base)
        fire(0)

        def pair_body(bb, carry):
            for par in range(2):
                b = b_base + bb * 2 + par
                drain(par)

                @pl.when(bb * 2 + par + 1 < b_per_w)
                def _():
                    fetch_idx(1 - par, b + 1)
                    fire(1 - par)

                process(par, b)
            return carry

        lax.fori_loop(0, b_per_w // 2, pair_body, 0)

    return k


def kernel(x, emb):
    b, s = x.shape
    v, d = emb.shape
    emb_p = jnp.pad(emb, ((0, 0), (0, LANES - d)))
    xf = x.reshape(-1).astype(jnp.int32)
    return _make_gather(b, s, d)(emb_p.astype(jnp.float32), xf)


# speedup vs baseline: 1.1827x; 1.1827x over previous
import functools

import jax
import jax.numpy as jnp
from jax import lax
from jax.experimental import pallas as pl
from jax.experimental.pallas import tpu as pltpu
from jax.experimental.pallas import tpu_sc as plsc

NC = 2
NS = 16
NW = NC * NS

LANES = 128
CB = 128
NRING = 5


@functools.lru_cache(maxsize=None)
def _make_gather(n_b: int, seq: int, dim: int):
    import math

    blk_tok = math.lcm(seq, CB)
    n_bl = blk_tok // seq
    n_ch = blk_tok // CB
    assert n_b % NW == 0
    b_per_w = n_b // NW
    assert b_per_w % n_bl == 0
    blocks_per_w = b_per_w // n_bl

    sched, first_need = [], []
    seen = set()
    for bl in range(n_bl):
        t0, t1 = seq * bl, seq * (bl + 1)
        segs = [
            (c, max(t0, CB * c), min(t1, CB * (c + 1)))
            for c in range(t0 // CB, (t1 - 1) // CB + 1)
        ]
        sched.append(segs)
        fresh = [c for c, _, _ in segs if c not in seen]
        seen.update(fresh)
        first_need.append(fresh)
    prologue_fire = first_need[0] + first_need[1]
    assert len(prologue_fire) <= NRING

    n_vec = dim // 16
    tail = dim - 16 * n_vec
    t_off = dim - 16

    mesh = plsc.VectorSubcoreMesh(core_axis_name="c", subcore_axis_name="s")

    @functools.partial(
        pl.kernel,
        mesh=mesh,
        out_type=jax.ShapeDtypeStruct((n_b, seq, dim), jnp.float32),
        scratch_types=[
            pltpu.VMEM((blk_tok,), jnp.int32),
            pltpu.VMEM((NRING, CB, LANES), jnp.float32),
            pltpu.VMEM((seq, dim), jnp.float32),
            pltpu.SemaphoreType.DMA,
        ],
        compiler_params=pltpu.CompilerParams(use_tc_tiling_on_sc=True),
    )
    def k(emb_hbm, xf_hbm, out_hbm, idx_v, rows_v, comp_v, sem_g):
        wid = lax.axis_index("s") * NC + lax.axis_index("c")
        b_base = wid * b_per_w

        def chunk_refs(c):
            return (emb_hbm.at[idx_v.at[pl.ds(c * CB, CB)]],
                    rows_v.at[c % NRING])

        def block_body(blk, carry):
            tok0 = (b_base + blk * n_bl) * seq
            pltpu.sync_copy(xf_hbm.at[pl.ds(tok0, blk_tok)], idx_v)
            for c in prologue_fire:
                src, dst = chunk_refs(c)
                pltpu.async_copy(src, dst, sem_g)

            for bl in range(n_bl):
                for c in first_need[bl]:
                    src, dst = chunk_refs(c)
                    pltpu.make_async_copy(src, dst, sem_g).wait()

                for c, lo, hi in sched[bl]:
                    ring = c % NRING
                    r_sh = c * CB
                    d_sh = bl * seq

                    def seg_body(t, carry2, ring=ring, r_sh=r_sh, d_sh=d_sh):
                        r = t - r_sh
                        d = t - d_sh
                        for v in range(n_vec):
                            comp_v[d, pl.ds(16 * v, 16)] = rows_v[
                                ring, r, pl.ds(16 * v, 16)
                            ]
                        if tail:
                            comp_v[d, pl.ds(t_off, 16)] = rows_v[
                                ring, r, pl.ds(t_off, 16)
                            ]
                        return carry2

                    lax.fori_loop(lo, hi, seg_body, 0)

                if bl + 2 < n_bl:
                    for c in first_need[bl + 2]:
                        src, dst = chunk_refs(c)
                        pltpu.async_copy(src, dst, sem_g)

                pltpu.sync_copy(
                    comp_v, out_hbm.at[b_base + blk * n_bl + bl]
                )
            return carry

        lax.fori_loop(0, blocks_per_w, block_body, 0)

    return k


def kernel(x, emb):
    b, s = x.shape
    v, d = emb.shape
    emb_p = jnp.pad(emb, ((0, 0), (0, LANES - d)))
    xf = x.reshape(-1).astype(jnp.int32)
    return _make_gather(b, s, d)(emb_p.astype(jnp.float32), xf)


# speedup vs baseline: 1.4620x; 1.2362x over previous
import functools

import jax
import jax.numpy as jnp
from jax import lax
from jax.experimental import pallas as pl
from jax.experimental.pallas import tpu as pltpu
from jax.experimental.pallas import tpu_sc as plsc

NC = 2
NS = 16
NW = NC * NS

BATCH = 128
SUPER = 8
GROUP = 2


@functools.lru_cache(maxsize=None)
def _make_gather(n_rows: int, dim: int, lanes: int):
    assert n_rows % (NW * SUPER) == 0
    rows_per_w = n_rows // NW
    n_super = rows_per_w // SUPER

    n_vec = dim // 16
    tail = dim - 16 * n_vec
    t_off = dim - 16

    mesh = plsc.VectorSubcoreMesh(core_axis_name="c", subcore_axis_name="s")

    @functools.partial(
        pl.kernel,
        mesh=mesh,
        out_type=jax.ShapeDtypeStruct((n_rows * BATCH, dim), jnp.float32),
        scratch_types=[
            pltpu.VMEM((2, SUPER, BATCH), jnp.int32),
            pltpu.VMEM((2, GROUP, BATCH, lanes), jnp.float32),
            pltpu.VMEM((GROUP * BATCH, dim), jnp.float32),
            pltpu.SemaphoreType.DMA,
        ],
        compiler_params=pltpu.CompilerParams(use_tc_tiling_on_sc=True),
    )
    def k(emb_hbm, xg_hbm, out_hbm, idx_v, rows_v, comp_v, sem_g):
        wid = lax.axis_index("s") * NC + lax.axis_index("c")
        row_base = wid * rows_per_w

        def fetch_idx(sb, sg):
            pltpu.sync_copy(
                xg_hbm.at[pl.ds(row_base + sg * SUPER, SUPER)], idx_v.at[sb]
            )

        def fire(buf, sb, jj):
            for j in range(GROUP):
                pltpu.async_copy(
                    emb_hbm.at[idx_v.at[sb, jj + j]], rows_v.at[buf, j], sem_g
                )

        def drain(buf, sb, jj):
            for j in range(GROUP):
                pltpu.make_async_copy(
                    emb_hbm.at[idx_v.at[sb, jj + j]], rows_v.at[buf, j], sem_g
                ).wait()

        def process(buf, g):
            for j in range(GROUP):

                def row_body(r, carry, j=j):
                    d = j * BATCH + r
                    for v in range(n_vec):
                        comp_v[d, pl.ds(16 * v, 16)] = rows_v[
                            buf, j, r, pl.ds(16 * v, 16)
                        ]
                    if tail:
                        comp_v[d, pl.ds(t_off, 16)] = rows_v[
                            buf, j, r, pl.ds(t_off, 16)
                        ]
                    return carry

                lax.fori_loop(0, BATCH, row_body, 0)

            pltpu.sync_copy(
                comp_v,
                out_hbm.at[pl.ds((row_base + g * GROUP) * BATCH,
                                 GROUP * BATCH)],
            )

        n_grp = SUPER // GROUP

        fetch_idx(0, 0)
        fire(0, 0, 0)

        def super_body(sg, carry):
            sb = lax.rem(sg, 2)

            @pl.when(sg + 1 < n_super)
            def _():
                fetch_idx(1 - sb, sg + 1)

            for i in range(n_grp):
                b = i % 2
                drain(b, sb, i * GROUP)
                if i + 1 < n_grp:
                    fire(1 - b, sb, (i + 1) * GROUP)
                else:

                    @pl.when(sg + 1 < n_super)
                    def _():
                        fire(1 - b, 1 - sb, 0)

                process(b, sg * n_grp + i)
            return carry

        lax.fori_loop(0, n_super, super_body, 0)

    return k


def kernel(x, emb):
    b, s = x.shape
    v, d = emb.shape
    n = b * s
    assert n % BATCH == 0
    lanes = 128
    emb_p = jnp.pad(emb, ((0, 0), (0, lanes - d)))
    xg = x.reshape(n // BATCH, BATCH).astype(jnp.int32)
    out = _make_gather(n // BATCH, d, lanes)(emb_p.astype(jnp.float32), xg)
    return out.reshape(b, s, d)
